# baseline (speedup 1.0000x reference)
import functools

import jax
import jax.numpy as jnp
from jax import lax
from jax.experimental import pallas as pl
from jax.experimental.pallas import tpu as pltpu
from jax.experimental.pallas import tpu_sc as plsc

B = 16384
HIST = 50
D = 32
NC = 2
NS = 16
NW = NC * NS
PB = B // NW
C = 16
NCH = PB // C
RPC = C * HIST
GSUB = 80
NSUB = RPC // GSUB
AVG_RATING = 3.0


_GDN = lax.GatherDimensionNumbers(
    offset_dims=(), collapsed_slice_dims=(0,), start_index_map=(0,))


def _permute(x, idx):
    return lax.gather(x, idx[:, None], _GDN, (1,),
                      mode=lax.GatherScatterMode.PROMISE_IN_BOUNDS)


def _hsum(x, iota):
    for sh in (1, 2, 4, 8):
        x = x + _permute(x, iota ^ sh)
    return x


def _rsqrt(x):
    y = (0.5 * jnp.where(x >= 4.0, 0.5, 1.0)
         * jnp.where(x >= 16.0, 0.5, 1.0))
    for _ in range(6):
        y = y * (1.5 - 0.5 * x * y * y)
    return y


@functools.partial(
    pl.kernel,
    out_type=(
        jax.ShapeDtypeStruct((B,), jnp.float32),
        jax.ShapeDtypeStruct((B,), jnp.float32),
        jax.ShapeDtypeStruct((B,), jnp.float32),
    ),
    mesh=plsc.VectorSubcoreMesh(core_axis_name="c", subcore_axis_name="s"),
    compiler_params=pltpu.CompilerParams(use_tc_tiling_on_sc=False),
    scratch_types=[
        pltpu.VMEM((2, NSUB, GSUB), jnp.int32),
        pltpu.VMEM((2, RPC), jnp.int32),
        pltpu.VMEM((2 * RPC, D), jnp.float32),
        pltpu.VMEM((2, C), jnp.int32),
        pltpu.VMEM((2, C), jnp.int32),
        pltpu.VMEM((2 * C, D), jnp.float32),
        pltpu.VMEM((2 * C, D), jnp.float32),
        pltpu.VMEM((2, C), jnp.float32),
        pltpu.VMEM((2, C), jnp.float32),
        pltpu.VMEM((1, D), jnp.float32),
        pltpu.VMEM((PB,), jnp.float32),
        pltpu.VMEM((PB,), jnp.float32),
        pltpu.VMEM((PB,), jnp.float32),
        pltpu.SemaphoreType.DMA,
        pltpu.SemaphoreType.DMA,
    ],
)
def _svdpp(user_h, item_h, simf_h, ub_h, ib_h, iq_h, up_h, iy_h,
           out_h, ubo_h, ibo_h,
           g2, sflat, rows, uidx, iidx, upc, iqc, ubc, ibc,
           y0, outv, ubov, ibov, sem_r, sem_s):
    wid = lax.axis_index("s") * NC + lax.axis_index("c")
    base = wid * PB
    iota = lax.iota(jnp.int32, 16)
    mtail = iota >= 14

    pltpu.sync_copy(iy_h.at[pl.ds(0, 1)], y0)
    y00 = y0[0, pl.ds(0, 16)]
    y01 = y0[0, pl.ds(16, 16)]

    def stage_and_fire(g, ph):
        cb = pl.multiple_of(base + g * C, C)
        pltpu.sync_copy(simf_h.at[pl.ds(pl.multiple_of(cb * HIST, RPC), RPC)],
                        sflat.at[ph])
        pltpu.sync_copy(user_h.at[pl.ds(cb, C)], uidx.at[ph])
        pltpu.sync_copy(item_h.at[pl.ds(cb, C)], iidx.at[ph])
        for j in range(RPC // 16):
            p = j * 16
            g2[ph, p // GSUB, pl.ds(p % GSUB, 16)] = sflat[ph, pl.ds(p, 16)]
        for j in range(NSUB):
            pltpu.async_copy(
                iy_h.at[g2.at[ph, j]],
                rows.at[pl.ds(ph * RPC + j * GSUB, GSUB), :], sem_r)
        pltpu.async_copy(up_h.at[uidx.at[ph]],
                         upc.at[pl.ds(ph * C, C), :], sem_s)
        pltpu.async_copy(iq_h.at[iidx.at[ph]],
                         iqc.at[pl.ds(ph * C, C), :], sem_s)
        pltpu.async_copy(ub_h.at[uidx.at[ph]], ubc.at[ph], sem_s)
        pltpu.async_copy(ib_h.at[iidx.at[ph]], ibc.at[ph], sem_s)

    def drain():
        for _ in range(NSUB):
            pltpu.make_async_copy(iy_h.at[pl.ds(0, GSUB)],
                                  rows.at[pl.ds(0, GSUB), :], sem_r).wait()
        pltpu.make_async_copy(up_h.at[pl.ds(0, C)],
                              upc.at[pl.ds(0, C), :], sem_s).wait()
        pltpu.make_async_copy(iq_h.at[pl.ds(0, C)],
                              iqc.at[pl.ds(0, C), :], sem_s).wait()
        pltpu.make_async_copy(ub_h.at[pl.ds(0, C)], ubc.at[0], sem_s).wait()
        pltpu.make_async_copy(ib_h.at[pl.ds(0, C)], ibc.at[0], sem_s).wait()

    def compute(g, ph):
        cnt = jnp.zeros((16,), jnp.float32)
        for b in range(C):
            p = b * HIST
            v0 = sflat[ph, pl.ds(p, 16)]
            v1 = sflat[ph, pl.ds(p + 16, 16)]
            v2 = sflat[ph, pl.ds(p + 32, 16)]
            v3 = sflat[ph, pl.ds(p + 34, 16)]
            z = (jnp.where(v0 == 0, 1.0, 0.0)
                 + jnp.where(v1 == 0, 1.0, 0.0)
                 + jnp.where(v2 == 0, 1.0, 0.0)
                 + jnp.where((v3 == 0) & mtail, 1.0, 0.0))
            cnt = jnp.where(iota == b, _hsum(z, iota), cnt)
        neff = 50.0 - cnt
        inv = 1.0 / (neff * _rsqrt(neff) + 1e-13)
        inv = jnp.where(neff == 0.0, 0.0, inv)

        tot = jnp.zeros((16,), jnp.float32)
        for b in range(C):
            fb = jnp.full((16,), b, jnp.int32)
            a0 = jnp.zeros((16,), jnp.float32)
            a1 = jnp.zeros((16,), jnp.float32)
            for n in range(HIST):
                r = ph * RPC + b * HIST + n
                a0 = a0 + rows[r, pl.ds(0, 16)]
                a1 = a1 + rows[r, pl.ds(16, 16)]
            c0 = _permute(cnt, fb)
            ivn = _permute(inv, fb)
            s0 = (a0 - c0 * y00) * ivn
            s1 = (a1 - c0 * y01) * ivn
            u0 = upc[ph * C + b, pl.ds(0, 16)]
            u1 = upc[ph * C + b, pl.ds(16, 16)]
            q0 = iqc[ph * C + b, pl.ds(0, 16)]
            q1 = iqc[ph * C + b, pl.ds(16, 16)]
            prod = (u0 + s0) * q0 + (u1 + s1) * q1
            tot = jnp.where(iota == b, _hsum(prod, iota), tot)

        ubv = ubc[ph]
        ibv = ibc[ph]
        off = g * C
        ubov[pl.ds(off, C)] = ubv
        ibov[pl.ds(off, C)] = ibv
        outv[pl.ds(off, C)] = AVG_RATING + ubv + ibv + tot

    stage_and_fire(0, 0)

    def pair(i, carry):
        g = i * 2
        drain()
        stage_and_fire(g + 1, 1)
        compute(g, 0)
        drain()

        @pl.when(i < NCH // 2 - 1)
        def _():
            stage_and_fire(g + 2, 0)

        compute(g + 1, 1)
        return carry

    lax.fori_loop(0, NCH // 2, pair, 0)

    pltpu.sync_copy(outv, out_h.at[pl.ds(base, PB)])
    pltpu.sync_copy(ubov, ubo_h.at[pl.ds(base, PB)])
    pltpu.sync_copy(ibov, ibo_h.at[pl.ds(base, PB)])


def kernel(user, item, similar_implicit, user_bias, item_bias, item_q,
           user_p, item_y):
    simf = similar_implicit.reshape(B * HIST)
    out, ub, ib = _svdpp(user, item, simf, user_bias, item_bias,
                         item_q, user_p, item_y)
    return (out, ub, ib)


# speedup vs baseline: 1.0126x; 1.0126x over previous
import functools

import jax
import jax.numpy as jnp
from jax import lax
from jax.experimental import pallas as pl
from jax.experimental.pallas import tpu as pltpu
from jax.experimental.pallas import tpu_sc as plsc

B = 16384
HIST = 50
D = 32
NC = 2
NS = 16
NW = NC * NS
PB = B // NW
C = 16
NCH = PB // C
RPC = C * HIST
GSUB = 80
NSUB = RPC // GSUB
AVG_RATING = 3.0


_GDN = lax.GatherDimensionNumbers(
    offset_dims=(), collapsed_slice_dims=(0,), start_index_map=(0,))


def _permute(x, idx):
    return lax.gather(x, idx[:, None], _GDN, (1,),
                      mode=lax.GatherScatterMode.PROMISE_IN_BOUNDS)


def _hsum(x, iota):
    for sh in (1, 2, 4, 8):
        x = x + _permute(x, iota ^ sh)
    return x


def _rsqrt(x):
    y = (0.5 * jnp.where(x >= 4.0, 0.5, 1.0)
         * jnp.where(x >= 16.0, 0.5, 1.0))
    for _ in range(6):
        y = y * (1.5 - 0.5 * x * y * y)
    return y


@functools.partial(
    pl.kernel,
    out_type=(
        jax.ShapeDtypeStruct((B,), jnp.float32),
        jax.ShapeDtypeStruct((B,), jnp.float32),
        jax.ShapeDtypeStruct((B,), jnp.float32),
    ),
    mesh=plsc.VectorSubcoreMesh(core_axis_name="c", subcore_axis_name="s"),
    compiler_params=pltpu.CompilerParams(use_tc_tiling_on_sc=False),
    scratch_types=[
        pltpu.VMEM((2, NSUB, GSUB), jnp.int32),
        pltpu.VMEM((2, RPC), jnp.int32),
        pltpu.VMEM((2 * RPC, D), jnp.float32),
        pltpu.VMEM((PB // 128, 128), jnp.int32),
        pltpu.VMEM((PB // 128, 128), jnp.int32),
        pltpu.VMEM((2 * C, D), jnp.float32),
        pltpu.VMEM((2 * C, D), jnp.float32),
        pltpu.VMEM((PB,), jnp.float32),
        pltpu.VMEM((PB,), jnp.float32),
        pltpu.VMEM((1, D), jnp.float32),
        pltpu.VMEM((PB,), jnp.float32),
        pltpu.VMEM((PB,), jnp.float32),
        pltpu.VMEM((PB,), jnp.float32),
        pltpu.SemaphoreType.DMA,
        pltpu.SemaphoreType.DMA,
    ],
)
def _svdpp(user2_h, item2_h, simf_h, ub_h, ib_h, iq_h, up_h, iy_h,
           out_h, ubo_h, ibo_h,
           g2, sflat, rows, uidxw, iidxw, upc, iqc, ubw, ibw,
           y0, outv, ubov, ibov, sem_r, sem_s):
    wid = lax.axis_index("s") * NC + lax.axis_index("c")
    base = wid * PB
    iota = lax.iota(jnp.int32, 16)
    mtail = iota >= 14

    pltpu.sync_copy(iy_h.at[pl.ds(0, 1)], y0)
    y00 = y0[0, pl.ds(0, 16)]
    y01 = y0[0, pl.ds(16, 16)]

    nrow = PB // 128
    pltpu.sync_copy(user2_h.at[pl.ds(pl.multiple_of(wid * nrow, nrow), nrow)],
                    uidxw)
    pltpu.sync_copy(item2_h.at[pl.ds(pl.multiple_of(wid * nrow, nrow), nrow)],
                    iidxw)
    bs = []
    for j in range(nrow):
        bs.append(pltpu.async_copy(ub_h.at[uidxw.at[j]],
                                   ubw.at[pl.ds(j * 128, 128)], sem_s))
        bs.append(pltpu.async_copy(ib_h.at[iidxw.at[j]],
                                   ibw.at[pl.ds(j * 128, 128)], sem_s))
    for cp in bs:
        cp.wait()

    def stage_and_fire(g, ph):
        cb = pl.multiple_of(base + g * C, C)
        pltpu.sync_copy(simf_h.at[pl.ds(pl.multiple_of(cb * HIST, RPC), RPC)],
                        sflat.at[ph])
        for j in range(RPC // 16):
            p = j * 16
            g2[ph, p // GSUB, pl.ds(p % GSUB, 16)] = sflat[ph, pl.ds(p, 16)]
        for j in range(NSUB):
            pltpu.async_copy(
                iy_h.at[g2.at[ph, j]],
                rows.at[pl.ds(ph * RPC + j * GSUB, GSUB), :], sem_r)
        uslc = uidxw.at[g // 8, pl.ds((g % 8) * C, C)]
        islc = iidxw.at[g // 8, pl.ds((g % 8) * C, C)]
        pltpu.async_copy(up_h.at[uslc], upc.at[pl.ds(ph * C, C), :], sem_s)
        pltpu.async_copy(iq_h.at[islc], iqc.at[pl.ds(ph * C, C), :], sem_s)

    def drain():
        pltpu.make_async_copy(iy_h.at[pl.ds(0, RPC)],
                              rows.at[pl.ds(0, RPC), :], sem_r).wait()
        pltpu.make_async_copy(up_h.at[pl.ds(0, 2 * C)],
                              upc.at[pl.ds(0, 2 * C), :], sem_s).wait()

    def compute(g, ph):
        cnt = jnp.zeros((16,), jnp.float32)
        for b in range(C):
            p = b * HIST
            v0 = sflat[ph, pl.ds(p, 16)]
            v1 = sflat[ph, pl.ds(p + 16, 16)]
            v2 = sflat[ph, pl.ds(p + 32, 16)]
            v3 = sflat[ph, pl.ds(p + 34, 16)]
            z = (jnp.where(v0 == 0, 1.0, 0.0)
                 + jnp.where(v1 == 0, 1.0, 0.0)
                 + jnp.where(v2 == 0, 1.0, 0.0)
                 + jnp.where((v3 == 0) & mtail, 1.0, 0.0))
            cnt = jnp.where(iota == b, _hsum(z, iota), cnt)
        neff = 50.0 - cnt
        inv = 1.0 / (neff * _rsqrt(neff) + 1e-13)
        inv = jnp.where(neff == 0.0, 0.0, inv)

        tot = jnp.zeros((16,), jnp.float32)
        for b in range(C):
            fb = jnp.full((16,), b, jnp.int32)
            a0 = jnp.zeros((16,), jnp.float32)
            a1 = jnp.zeros((16,), jnp.float32)
            for n in range(HIST):
                r = ph * RPC + b * HIST + n
                a0 = a0 + rows[r, pl.ds(0, 16)]
                a1 = a1 + rows[r, pl.ds(16, 16)]
            c0 = _permute(cnt, fb)
            ivn = _permute(inv, fb)
            s0 = (a0 - c0 * y00) * ivn
            s1 = (a1 - c0 * y01) * ivn
            u0 = upc[ph * C + b, pl.ds(0, 16)]
            u1 = upc[ph * C + b, pl.ds(16, 16)]
            q0 = iqc[ph * C + b, pl.ds(0, 16)]
            q1 = iqc[ph * C + b, pl.ds(16, 16)]
            prod = (u0 + s0) * q0 + (u1 + s1) * q1
            tot = jnp.where(iota == b, _hsum(prod, iota), tot)

        off = g * C
        ubv = ubw[pl.ds(off, C)]
        ibv = ibw[pl.ds(off, C)]
        ubov[pl.ds(off, C)] = ubv
        ibov[pl.ds(off, C)] = ibv
        outv[pl.ds(off, C)] = AVG_RATING + ubv + ibv + tot

    stage_and_fire(0, 0)

    def pair(i, carry):
        g = i * 2
        drain()
        stage_and_fire(g + 1, 1)
        compute(g, 0)
        drain()

        @pl.when(i < NCH // 2 - 1)
        def _():
            stage_and_fire(g + 2, 0)

        compute(g + 1, 1)
        return carry

    lax.fori_loop(0, NCH // 2, pair, 0)

    pltpu.sync_copy(outv, out_h.at[pl.ds(base, PB)])
    pltpu.sync_copy(ubov, ubo_h.at[pl.ds(base, PB)])
    pltpu.sync_copy(ibov, ibo_h.at[pl.ds(base, PB)])


def kernel(user, item, similar_implicit, user_bias, item_bias, item_q,
           user_p, item_y):
    simf = similar_implicit.reshape(B * HIST)
    out, ub, ib = _svdpp(user.reshape(B // 128, 128),
                         item.reshape(B // 128, 128), simf,
                         user_bias, item_bias, item_q, user_p, item_y)
    return (out, ub, ib)


# speedup vs baseline: 1.0437x; 1.0307x over previous
import functools

import jax
import jax.numpy as jnp
from jax import lax
from jax.experimental import pallas as pl
from jax.experimental.pallas import tpu as pltpu
from jax.experimental.pallas import tpu_sc as plsc

B = 16384
HIST = 50
D = 32
NC = 2
NS = 16
NW = NC * NS
PB = B // NW
C = 16
NCH = PB // C
RPC = C * HIST
GSUB = 80
NSUB = RPC // GSUB
AVG_RATING = 3.0


_GDN = lax.GatherDimensionNumbers(
    offset_dims=(), collapsed_slice_dims=(0,), start_index_map=(0,))


def _permute(x, idx):
    return lax.gather(x, idx[:, None], _GDN, (1,),
                      mode=lax.GatherScatterMode.PROMISE_IN_BOUNDS)


def _hsum(x, iota):
    for sh in (1, 2, 4, 8):
        x = x + _permute(x, iota ^ sh)
    return x


def _rsqrt(x):
    y = (0.5 * jnp.where(x >= 4.0, 0.5, 1.0)
         * jnp.where(x >= 16.0, 0.5, 1.0))
    for _ in range(6):
        y = y * (1.5 - 0.5 * x * y * y)
    return y


@functools.partial(
    pl.kernel,
    out_type=(
        jax.ShapeDtypeStruct((B,), jnp.float32),
        jax.ShapeDtypeStruct((B,), jnp.float32),
        jax.ShapeDtypeStruct((B,), jnp.float32),
    ),
    mesh=plsc.VectorSubcoreMesh(core_axis_name="c", subcore_axis_name="s"),
    compiler_params=pltpu.CompilerParams(use_tc_tiling_on_sc=False),
    scratch_types=[
        pltpu.VMEM((2, NSUB, GSUB), jnp.int32),
        pltpu.VMEM((2, RPC), jnp.int32),
        pltpu.VMEM((2 * RPC, D), jnp.float32),
        pltpu.VMEM((PB // 128, 128), jnp.int32),
        pltpu.VMEM((PB // 128, 128), jnp.int32),
        pltpu.VMEM((2 * C, D), jnp.float32),
        pltpu.VMEM((2 * C, D), jnp.float32),
        pltpu.VMEM((PB,), jnp.float32),
        pltpu.VMEM((PB,), jnp.float32),
        pltpu.VMEM((2, 16), jnp.float32),
        pltpu.VMEM((2, 16), jnp.float32),
        pltpu.VMEM((1, D), jnp.float32),
        pltpu.VMEM((PB,), jnp.float32),
        pltpu.VMEM((PB,), jnp.float32),
        pltpu.VMEM((PB,), jnp.float32),
        pltpu.SemaphoreType.DMA,
        pltpu.SemaphoreType.DMA,
        pltpu.SemaphoreType.DMA,
    ],
)
def _svdpp(user2_h, item2_h, simf_h, ub_h, ib_h, iq_h, up_h, iy_h,
           out_h, ubo_h, ibo_h,
           g2, sflat, rows, uidxw, iidxw, upc, iqc, ubw, ibw,
           cnt_r, inv_r, y0, outv, ubov, ibov, sem_r, sem_s, sem_i):
    wid = lax.axis_index("s") * NC + lax.axis_index("c")
    base = wid * PB
    iota = lax.iota(jnp.int32, 16)
    mtail = iota >= 14

    pltpu.sync_copy(iy_h.at[pl.ds(0, 1)], y0)
    y00 = y0[0, pl.ds(0, 16)]
    y01 = y0[0, pl.ds(16, 16)]

    nrow = PB // 128
    pltpu.sync_copy(user2_h.at[pl.ds(pl.multiple_of(wid * nrow, nrow), nrow)],
                    uidxw)
    pltpu.sync_copy(item2_h.at[pl.ds(pl.multiple_of(wid * nrow, nrow), nrow)],
                    iidxw)
    bs = []
    for j in range(nrow):
        bs.append(pltpu.async_copy(ub_h.at[uidxw.at[j]],
                                   ubw.at[pl.ds(j * 128, 128)], sem_s))
        bs.append(pltpu.async_copy(ib_h.at[iidxw.at[j]],
                                   ibw.at[pl.ds(j * 128, 128)], sem_s))
    for cp in bs:
        cp.wait()

    def prefetch_sflat(g, ph):
        cb = pl.multiple_of(base + g * C, C)
        pltpu.async_copy(
            simf_h.at[pl.ds(pl.multiple_of(cb * HIST, RPC), RPC)],
            sflat.at[ph], sem_i)

    def stage_and_fire(g, ph):
        pltpu.make_async_copy(simf_h.at[pl.ds(0, RPC)], sflat.at[ph],
                              sem_i).wait()
        for j in range(RPC // 16):
            p = j * 16
            g2[ph, p // GSUB, pl.ds(p % GSUB, 16)] = sflat[ph, pl.ds(p, 16)]
        for j in range(NSUB):
            pltpu.async_copy(
                iy_h.at[g2.at[ph, j]],
                rows.at[pl.ds(ph * RPC + j * GSUB, GSUB), :], sem_r)
        uslc = uidxw.at[g // 8, pl.ds((g % 8) * C, C)]
        islc = iidxw.at[g // 8, pl.ds((g % 8) * C, C)]
        pltpu.async_copy(up_h.at[uslc], upc.at[pl.ds(ph * C, C), :], sem_s)
        pltpu.async_copy(iq_h.at[islc], iqc.at[pl.ds(ph * C, C), :], sem_s)
        cnt = jnp.zeros((16,), jnp.float32)
        for b in range(C):
            p = b * HIST
            v0 = sflat[ph, pl.ds(p, 16)]
            v1 = sflat[ph, pl.ds(p + 16, 16)]
            v2 = sflat[ph, pl.ds(p + 32, 16)]
            v3 = sflat[ph, pl.ds(p + 34, 16)]
            z = (jnp.where(v0 == 0, 1.0, 0.0)
                 + jnp.where(v1 == 0, 1.0, 0.0)
                 + jnp.where(v2 == 0, 1.0, 0.0)
                 + jnp.where((v3 == 0) & mtail, 1.0, 0.0))
            cnt = jnp.where(iota == b, _hsum(z, iota), cnt)
        neff = 50.0 - cnt
        inv = 1.0 / (neff * _rsqrt(neff) + 1e-13)
        cnt_r[ph] = cnt
        inv_r[ph] = jnp.where(neff == 0.0, 0.0, inv)

    def drain():
        pltpu.make_async_copy(iy_h.at[pl.ds(0, RPC)],
                              rows.at[pl.ds(0, RPC), :], sem_r).wait()
        pltpu.make_async_copy(up_h.at[pl.ds(0, 2 * C)],
                              upc.at[pl.ds(0, 2 * C), :], sem_s).wait()

    def compute(g, ph):
        cnt = cnt_r[ph]
        inv = inv_r[ph]
        tot = jnp.zeros((16,), jnp.float32)
        for b in range(C):
            fb = jnp.full((16,), b, jnp.int32)
            a0 = jnp.zeros((16,), jnp.float32)
            a1 = jnp.zeros((16,), jnp.float32)
            for n in range(HIST):
                r = ph * RPC + b * HIST + n
                a0 = a0 + rows[r, pl.ds(0, 16)]
                a1 = a1 + rows[r, pl.ds(16, 16)]
            c0 = _permute(cnt, fb)
            ivn = _permute(inv, fb)
            s0 = (a0 - c0 * y00) * ivn
            s1 = (a1 - c0 * y01) * ivn
            u0 = upc[ph * C + b, pl.ds(0, 16)]
            u1 = upc[ph * C + b, pl.ds(16, 16)]
            q0 = iqc[ph * C + b, pl.ds(0, 16)]
            q1 = iqc[ph * C + b, pl.ds(16, 16)]
            prod = (u0 + s0) * q0 + (u1 + s1) * q1
            tot = jnp.where(iota == b, _hsum(prod, iota), tot)

        off = g * C
        ubv = ubw[pl.ds(off, C)]
        ibv = ibw[pl.ds(off, C)]
        ubov[pl.ds(off, C)] = ubv
        ibov[pl.ds(off, C)] = ibv
        outv[pl.ds(off, C)] = AVG_RATING + ubv + ibv + tot

    prefetch_sflat(0, 0)
    prefetch_sflat(1, 1)
    stage_and_fire(0, 0)

    def pair(i, carry):
        g = i * 2
        drain()

        @pl.when(i < NCH // 2 - 1)
        def _():
            prefetch_sflat(g + 2, 0)

        stage_and_fire(g + 1, 1)
        compute(g, 0)
        drain()

        @pl.when(i < NCH // 2 - 1)
        def _():
            prefetch_sflat(g + 3, 1)
            stage_and_fire(g + 2, 0)

        compute(g + 1, 1)
        return carry

    lax.fori_loop(0, NCH // 2, pair, 0)

    pltpu.sync_copy(outv, out_h.at[pl.ds(base, PB)])
    pltpu.sync_copy(ubov, ubo_h.at[pl.ds(base, PB)])
    pltpu.sync_copy(ibov, ibo_h.at[pl.ds(base, PB)])


def kernel(user, item, similar_implicit, user_bias, item_bias, item_q,
           user_p, item_y):
    simf = similar_implicit.reshape(B * HIST)
    out, ub, ib = _svdpp(user.reshape(B // 128, 128),
                         item.reshape(B // 128, 128), simf,
                         user_bias, item_bias, item_q, user_p, item_y)
    return (out, ub, ib)
